# baseline (speedup 1.0000x reference)
import functools

import jax
import jax.numpy as jnp
from jax import lax
from jax.experimental import pallas as pl
from jax.experimental.pallas import tpu as pltpu
from jax.experimental.pallas import tpu_sc as plsc

_BETA = 0.25
_K = 8192
_D = 256
_N = 8192
_TBLK = 1024
_KBLK = 1024


def _dist_body(z_ref, e_ref, idx_ref, loss_ref, sume_ref, en2_ref):
    i = pl.program_id(0)

    @pl.when(i == 0)
    def _():
        for kb in range(_K // _KBLK):
            e = e_ref[pl.ds(kb * _KBLK, _KBLK), :]
            sume_ref[0, pl.ds(kb * _KBLK, _KBLK)] = jnp.sum(e * e, axis=1)
            en2_ref[pl.ds(kb * _KBLK, _KBLK), :] = -2.0 * e

    z = jnp.transpose(z_ref[0], (1, 0))
    sumz = jnp.sum(z * z, axis=1, keepdims=True)

    run_min = None
    run_idx = None
    for kb in range(_K // _KBLK):
        en2 = en2_ref[pl.ds(kb * _KBLK, _KBLK), :]
        sume = sume_ref[0, pl.ds(kb * _KBLK, _KBLK)]
        mm2 = lax.dot_general(
            z, en2, (((1,), (1,)), ((), ())),
            preferred_element_type=jnp.float32,
        )
        d = sumz + mm2 + sume[None, :]
        m = jnp.min(d, axis=1, keepdims=True)
        iota = lax.broadcasted_iota(jnp.int32, (1, _KBLK), 1).astype(jnp.float32)
        bidx = jnp.min(jnp.where(d == m, iota, jnp.float32(_K)),
                       axis=1, keepdims=True)
        bidx = bidx + jnp.float32(kb * _KBLK)
        if run_min is None:
            run_min, run_idx = m, bidx
        else:
            better = m < run_min
            run_idx = jnp.where(better, bidx, run_idx)
            run_min = jnp.where(better, m, run_min)

    idx_ref[...] = jnp.reshape(run_idx[:, 0], (_TBLK // 128, 128)).astype(jnp.int32)
    part = jnp.sum(run_min)

    @pl.when(i == 0)
    def _():
        loss_ref[0, 0] = part

    @pl.when(i > 0)
    def _():
        loss_ref[0, 0] = loss_ref[0, 0] + part


def _distances_argmin(z3, emb):
    nrows_per_step = _TBLK // 128
    idx, losssum = pl.pallas_call(
        _dist_body,
        grid=(_N // _TBLK,),
        in_specs=[
            pl.BlockSpec((1, _D, _TBLK), lambda i: (i, 0, 0)),
            pl.BlockSpec((_K, _D), lambda i: (0, 0)),
        ],
        out_specs=[
            pl.BlockSpec((nrows_per_step, 128), lambda i: (i, 0)),
            pl.BlockSpec(memory_space=pltpu.SMEM),
        ],
        out_shape=[
            jax.ShapeDtypeStruct((_N // 128, 128), jnp.int32),
            jax.ShapeDtypeStruct((1, 1), jnp.float32),
        ],
        scratch_shapes=[pltpu.VMEM((1, _K), jnp.float32),
                        pltpu.VMEM((_K, _D), jnp.float32)],
    )(z3, emb)
    return idx, losssum


_NC, _NS = 2, 16
_NW = _NC * _NS
_CH = 128
_NROWS = _N // _CH
_CPW = _NROWS // _NW


@functools.cache
def _make_sc_gather():
    @functools.partial(
        pl.kernel,
        out_type=jax.ShapeDtypeStruct((_NROWS, _CH, _D), jnp.float32),
        mesh=plsc.VectorSubcoreMesh(core_axis_name="c", subcore_axis_name="s",
                                    num_cores=_NC, num_subcores=_NS),
        scratch_types=[
            pltpu.VMEM((_CPW, _CH), jnp.int32),
            pltpu.VMEM((_CPW, _CH, _D), jnp.float32),
            pltpu.SemaphoreType.DMA,
        ],
    )
    def _sc_gather(table_hbm, idx_hbm, out_hbm, idx_v, rows_v, sem):
        wid = lax.axis_index("s") * _NC + lax.axis_index("c")
        base = wid * _CPW
        pltpu.sync_copy(idx_hbm.at[pl.ds(base, _CPW)], idx_v)
        for j in range(_CPW):
            pltpu.async_copy(table_hbm.at[idx_v.at[j]], rows_v.at[j], sem).wait()
        pltpu.sync_copy(rows_v, out_hbm.at[pl.ds(base, _CPW)])

    return _sc_gather


def kernel(z, embedding_weight):
    B, D, H, W = z.shape
    z3 = z.reshape(B, D, H * W)

    idx2d, losssum = _distances_argmin(z3, embedding_weight)
    indices = idx2d.reshape(B, H, W)

    zq_rows = _make_sc_gather()(embedding_weight, idx2d)
    z_q = jnp.transpose(zq_rows.reshape(B, H, W, D), (0, 3, 1, 2))

    loss = (losssum[0, 0] / jnp.float32(_N * _D)) * jnp.float32(_BETA)
    return (z_q, indices, loss)


# speedup vs baseline: 1.0069x; 1.0069x over previous
import functools

import jax
import jax.numpy as jnp
from jax import lax
from jax.experimental import pallas as pl
from jax.experimental.pallas import tpu as pltpu
from jax.experimental.pallas import tpu_sc as plsc

_BETA = 0.25
_K = 8192
_D = 256
_N = 8192
_TBLK = 1024
_KBLK = 2048


def _dist_body(z_ref, e_ref, idx_ref, loss_ref, sume_ref, en2_ref, iota_ref):
    i = pl.program_id(0)

    @pl.when(i == 0)
    def _():
        for kb in range(_K // _KBLK):
            e = e_ref[pl.ds(kb * _KBLK, _KBLK), :]
            sume_ref[0, pl.ds(kb * _KBLK, _KBLK)] = jnp.sum(e * e, axis=1)
            en2_ref[pl.ds(kb * _KBLK, _KBLK), :] = -2.0 * e
            iota_ref[0, pl.ds(kb * _KBLK, _KBLK)] = (
                jnp.float32(kb * _KBLK)
                + lax.broadcasted_iota(jnp.int32, (1, _KBLK), 1).astype(jnp.float32)
            )[0]

    z = jnp.transpose(z_ref[0], (1, 0))
    sumz = jnp.sum(z * z, axis=1, keepdims=True)

    run_min = None
    run_idx = None
    for kb in range(_K // _KBLK):
        en2 = en2_ref[pl.ds(kb * _KBLK, _KBLK), :]
        sume = sume_ref[0, pl.ds(kb * _KBLK, _KBLK)]
        mm2 = lax.dot_general(
            z, en2, (((1,), (1,)), ((), ())),
            preferred_element_type=jnp.float32,
        )
        d = sumz + mm2 + sume[None, :]
        m = jnp.min(d, axis=1, keepdims=True)
        iota = iota_ref[0, pl.ds(kb * _KBLK, _KBLK)]
        bidx = jnp.min(jnp.where(d == m, iota[None, :], jnp.float32(_K)),
                       axis=1, keepdims=True)
        if run_min is None:
            run_min, run_idx = m, bidx
        else:
            better = m < run_min
            run_idx = jnp.where(better, bidx, run_idx)
            run_min = jnp.where(better, m, run_min)

    idx_ref[...] = jnp.reshape(run_idx[:, 0], (_TBLK // 128, 128)).astype(jnp.int32)
    part = jnp.sum(run_min)

    @pl.when(i == 0)
    def _():
        loss_ref[0, 0] = part

    @pl.when(i > 0)
    def _():
        loss_ref[0, 0] = loss_ref[0, 0] + part


def _distances_argmin(z3, emb):
    nrows_per_step = _TBLK // 128
    idx, losssum = pl.pallas_call(
        _dist_body,
        grid=(_N // _TBLK,),
        in_specs=[
            pl.BlockSpec((1, _D, _TBLK), lambda i: (i, 0, 0)),
            pl.BlockSpec((_K, _D), lambda i: (0, 0)),
        ],
        out_specs=[
            pl.BlockSpec((nrows_per_step, 128), lambda i: (i, 0)),
            pl.BlockSpec(memory_space=pltpu.SMEM),
        ],
        out_shape=[
            jax.ShapeDtypeStruct((_N // 128, 128), jnp.int32),
            jax.ShapeDtypeStruct((1, 1), jnp.float32),
        ],
        scratch_shapes=[pltpu.VMEM((1, _K), jnp.float32),
                        pltpu.VMEM((_K, _D), jnp.float32),
                        pltpu.VMEM((1, _K), jnp.float32)],
    )(z3, emb)
    return idx, losssum


_NC, _NS = 2, 16
_NW = _NC * _NS
_CH = 128
_NROWS = _N // _CH
_CPW = _NROWS // _NW


@functools.cache
def _make_sc_gather():
    @functools.partial(
        pl.kernel,
        out_type=jax.ShapeDtypeStruct((_NROWS, _CH, _D), jnp.float32),
        mesh=plsc.VectorSubcoreMesh(core_axis_name="c", subcore_axis_name="s",
                                    num_cores=_NC, num_subcores=_NS),
        scratch_types=[
            pltpu.VMEM((_CPW, _CH), jnp.int32),
            pltpu.VMEM((_CPW, _CH, _D), jnp.float32),
            pltpu.SemaphoreType.DMA,
        ],
    )
    def _sc_gather(table_hbm, idx_hbm, out_hbm, idx_v, rows_v, sem):
        wid = lax.axis_index("s") * _NC + lax.axis_index("c")
        base = wid * _CPW
        pltpu.sync_copy(idx_hbm.at[pl.ds(base, _CPW)], idx_v)
        for j in range(_CPW):
            pltpu.async_copy(table_hbm.at[idx_v.at[j]], rows_v.at[j], sem).wait()
        pltpu.sync_copy(rows_v, out_hbm.at[pl.ds(base, _CPW)])

    return _sc_gather


def kernel(z, embedding_weight):
    B, D, H, W = z.shape
    z3 = z.reshape(B, D, H * W)

    idx2d, losssum = _distances_argmin(z3, embedding_weight)
    indices = idx2d.reshape(B, H, W)

    zq_rows = _make_sc_gather()(embedding_weight, idx2d)
    z_q = jnp.transpose(zq_rows.reshape(B, H, W, D), (0, 3, 1, 2))

    loss = (losssum[0, 0] / jnp.float32(_N * _D)) * jnp.float32(_BETA)
    return (z_q, indices, loss)
